# Initial kernel scaffold; baseline (speedup 1.0000x reference)
import jax
import jax.numpy as jnp
from jax.experimental import pallas as pl


def kernel(x, edge_index, alpha1, alpha2, gamma, bias):
    raise NotImplementedError("write your pallas kernel here")


# speedup vs baseline: 51.0659x; 51.0659x over previous
import jax
import jax.numpy as jnp
from jax import lax
from jax.experimental import pallas as pl
from jax.experimental.pallas import tpu as pltpu
from jax.experimental.pallas import tpu_sc as plsc

NC = 2
NS = 16
W = 128
KJ = 16
CH = 4


def _sc_pass(xt, srcp, dstp, zeros, ones, *, np_, nbw, with_deg):
    g_iters = nbw // KJ
    slab = np_ // NS
    mesh = plsc.VectorSubcoreMesh(core_axis_name="c", subcore_axis_name="s")
    out_type = [jax.ShapeDtypeStruct((NC, np_, CH), jnp.float32)]
    if with_deg:
        out_type.append(jax.ShapeDtypeStruct((NC, np_, CH), jnp.float32))
    scratch = (
        [pltpu.VMEM_SHARED((np_, CH), jnp.float32),
         pltpu.VMEM_SHARED((np_, CH), jnp.float32)]
        + ([pltpu.VMEM_SHARED((np_, CH), jnp.float32)] if with_deg else [])
        + [pltpu.VMEM((KJ, W), jnp.int32),
           pltpu.VMEM((KJ, W), jnp.int32),
           pltpu.VMEM((KJ * W, CH), jnp.float32),
           pltpu.VMEM((W, CH), jnp.float32),
           pltpu.SemaphoreType.DMA,
           pltpu.SemaphoreType.DMA]
    )

    def body(xt_h, srcp_h, dstp_h, zeros_h, ones_h, *rest):
        if with_deg:
            (s_out, d_out, tab, acc, dacc, srcb, dstb, msg, ones_v,
             sem_g, sem_s) = rest
        else:
            (s_out, tab, acc, srcb, dstb, msg, ones_v,
             sem_g, sem_s) = rest
            d_out = dacc = None
        cid = lax.axis_index("c")
        sid = lax.axis_index("s")
        row0 = sid * slab
        pltpu.sync_copy(zeros_h, acc.at[pl.ds(row0, slab)])
        if with_deg:
            pltpu.sync_copy(zeros_h, dacc.at[pl.ds(row0, slab)])
            pltpu.sync_copy(ones_h, ones_v)
        pltpu.sync_copy(xt_h.at[pl.ds(row0, slab)], tab.at[pl.ds(row0, slab)])
        plsc.subcore_barrier()

        blk0 = (cid * NS + sid) * nbw

        def step(g, carry):
            row = blk0 + g * KJ
            pltpu.sync_copy(srcp_h.at[pl.ds(row, KJ)], srcb)
            pltpu.sync_copy(dstp_h.at[pl.ds(row, KJ)], dstb)
            cps = []
            for j in range(KJ):
                cps.append(pltpu.async_copy(
                    tab.at[srcb.at[j]], msg.at[pl.ds(j * W, W)], sem_g))
            for c in cps:
                c.wait()
            cps = []
            for j in range(KJ):
                cps.append(pltpu.async_copy(
                    msg.at[pl.ds(j * W, W)], acc.at[dstb.at[j]], sem_s,
                    add=True))
                if with_deg:
                    cps.append(pltpu.async_copy(
                        ones_v, dacc.at[srcb.at[j]], sem_s, add=True))
            for c in cps:
                c.wait()
            return carry

        lax.fori_loop(0, g_iters, step, 0)
        plsc.subcore_barrier()
        pltpu.sync_copy(acc.at[pl.ds(row0, slab)],
                        s_out.at[cid, pl.ds(row0, slab)])
        if with_deg:
            pltpu.sync_copy(dacc.at[pl.ds(row0, slab)],
                            d_out.at[cid, pl.ds(row0, slab)])

    f = pl.kernel(body, out_type=tuple(out_type), mesh=mesh,
                  scratch_types=scratch,
                  compiler_params=pltpu.CompilerParams(
                      use_tc_tiling_on_sc=False))
    return f(xt, srcp, dstp, zeros, ones)


def _combine(xf, saf, sbf, daf, dbf, params):
    m = xf.shape[0]

    def body(p_ref, x_ref, sa_ref, sb_ref, da_ref, db_ref, o_ref):
        a1 = p_ref[0]
        g = p_ref[1]
        b = p_ref[2]
        dp = jax.nn.sigmoid(g)
        sw = jnp.exp(a1)
        nw = sw * jnp.tanh(a1)
        ld = jnp.log(da_ref[...] + db_ref[...])
        s = sa_ref[...] + sb_ref[...]
        o_ref[...] = (sw * jnp.exp(dp * ld) * x_ref[...]
                      + nw * jnp.exp((dp - 1.0) * ld) * s + b)

    return pl.pallas_call(
        body,
        out_shape=jax.ShapeDtypeStruct((m, 128), jnp.float32),
        in_specs=[pl.BlockSpec(memory_space=pltpu.SMEM)]
        + [pl.BlockSpec(memory_space=pltpu.VMEM)] * 5,
        out_specs=pl.BlockSpec(memory_space=pltpu.VMEM),
    )(params, xf, saf, sbf, daf, dbf)


def kernel(x, edge_index, alpha1, alpha2, gamma, bias):
    t_ch, n = x.shape
    e = edge_index.shape[1]
    n_layers = alpha1.shape[0]
    np_ = (n // 2048 + 1) * 2048
    m = np_ * CH // 128
    nbw = (-(-e // (W * NC * NS * KJ))) * KJ
    ep = nbw * NC * NS * W
    npad = ep - e

    src = edge_index[0]
    dst = edge_index[1]
    pad_ids = n + (jnp.arange(npad, dtype=jnp.int32) % jnp.int32(np_ - n))
    srcp = jnp.concatenate([src, pad_ids]).reshape(-1, W)
    dstp = jnp.concatenate([dst, pad_ids]).reshape(-1, W)

    xt = jnp.pad(x.T, ((0, np_ - n), (0, 0)))
    zeros = jnp.zeros((np_ // NS, CH), jnp.float32)
    ones = jnp.ones((W, CH), jnp.float32)

    outf = xt.reshape(m, 128)
    da = db = None
    for i in range(n_layers):
        tab = outf.reshape(np_, CH)
        if i == 0:
            (sp, dp_) = _sc_pass(tab, srcp, dstp, zeros, ones,
                                 np_=np_, nbw=nbw, with_deg=True)
            da = dp_[0].reshape(m, 128)
            db = dp_[1].reshape(m, 128)
        else:
            (sp,) = _sc_pass(tab, srcp, dstp, zeros, ones,
                             np_=np_, nbw=nbw, with_deg=False)
        params = jnp.stack([alpha1[i, 0, 0], gamma[i, 0, 0], bias[i, 0, 0]])
        outf = _combine(outf, sp[0].reshape(m, 128), sp[1].reshape(m, 128),
                        da, db, params)
    return outf.reshape(np_, CH)[:n].T


# speedup vs baseline: 59.3750x; 1.1627x over previous
import jax
import jax.numpy as jnp
from jax import lax
from jax.experimental import pallas as pl
from jax.experimental.pallas import tpu as pltpu
from jax.experimental.pallas import tpu_sc as plsc

NC = 2
NS = 16
NW = NC * NS
W = 128
KJ = 16
CH = 4
LN2 = 0.6931471805599453

_L2C = (0.0439286278, -0.409475586, 1.61017755, -3.52021884, 5.06975632,
        -2.79415368)

_SC_PARAMS = pltpu.CompilerParams(use_tc_tiling_on_sc=False)


def _sc_pass(xt, srcw, dstw, zeros, ones, *, np_, nwin, with_deg):
    slab = np_ // NS
    base = nwin // NW
    rem = nwin % NW
    mesh = plsc.VectorSubcoreMesh(core_axis_name="c", subcore_axis_name="s")
    out_type = [jax.ShapeDtypeStruct((NC, np_, CH), jnp.float32)]
    if with_deg:
        out_type.append(jax.ShapeDtypeStruct((NC, np_, CH), jnp.float32))
    scratch = (
        [pltpu.VMEM_SHARED((np_, CH), jnp.float32),
         pltpu.VMEM_SHARED((np_, CH), jnp.float32)]
        + ([pltpu.VMEM_SHARED((np_, CH), jnp.float32)] if with_deg else [])
        + [pltpu.VMEM((KJ, W), jnp.int32),
           pltpu.VMEM((KJ, W), jnp.int32),
           pltpu.VMEM((KJ * W, CH), jnp.float32),
           pltpu.VMEM((W, CH), jnp.float32),
           pltpu.SemaphoreType.DMA,
           pltpu.SemaphoreType.DMA]
    )

    def body(xt_h, srcw_h, dstw_h, zeros_h, ones_h, *rest):
        if with_deg:
            (s_out, d_out, tab, acc, dacc, srcb, dstb, msg, ones_v,
             sem_g, sem_s) = rest
        else:
            (s_out, tab, acc, srcb, dstb, msg, ones_v,
             sem_g, sem_s) = rest
            d_out = dacc = None
        cid = lax.axis_index("c")
        sid = lax.axis_index("s")
        row0 = sid * slab
        pltpu.sync_copy(zeros_h, acc.at[pl.ds(row0, slab)])
        if with_deg:
            pltpu.sync_copy(zeros_h, dacc.at[pl.ds(row0, slab)])
            pltpu.sync_copy(ones_h, ones_v)
        pltpu.sync_copy(xt_h.at[pl.ds(row0, slab)], tab.at[pl.ds(row0, slab)])
        plsc.subcore_barrier()

        wid = cid * NS + sid
        wstart = wid * base + jnp.minimum(wid, rem)
        wcnt = base + jnp.where(wid < rem, 1, 0)
        nfull = wcnt // KJ

        def do_windows(rowbase, nj):
            cps = []
            for j in range(nj):
                cps.append(pltpu.async_copy(
                    tab.at[srcb.at[j]], msg.at[pl.ds(j * W, W)], sem_g))
            for c in cps:
                c.wait()
            cps = []
            for j in range(nj):
                cps.append(pltpu.async_copy(
                    msg.at[pl.ds(j * W, W)], acc.at[dstb.at[j]], sem_s,
                    add=True))
                if with_deg:
                    cps.append(pltpu.async_copy(
                        ones_v, dacc.at[srcb.at[j]], sem_s, add=True))
            for c in cps:
                c.wait()

        def step(g, carry):
            row = wstart + g * KJ
            pltpu.sync_copy(srcw_h.at[pl.ds(row, KJ)], srcb)
            pltpu.sync_copy(dstw_h.at[pl.ds(row, KJ)], dstb)
            do_windows(row, KJ)
            return carry

        lax.fori_loop(0, nfull, step, 0)

        def tail_step(t, carry):
            row = wstart + nfull * KJ + t
            pltpu.sync_copy(srcw_h.at[pl.ds(row, 1)], srcb.at[pl.ds(0, 1)])
            pltpu.sync_copy(dstw_h.at[pl.ds(row, 1)], dstb.at[pl.ds(0, 1)])
            do_windows(row, 1)
            return carry

        lax.fori_loop(0, wcnt - nfull * KJ, tail_step, 0)

        plsc.subcore_barrier()
        pltpu.sync_copy(acc.at[pl.ds(row0, slab)],
                        s_out.at[cid, pl.ds(row0, slab)])
        if with_deg:
            pltpu.sync_copy(dacc.at[pl.ds(row0, slab)],
                            d_out.at[cid, pl.ds(row0, slab)])

    f = pl.kernel(body, out_type=tuple(out_type), mesh=mesh,
                  scratch_types=scratch, compiler_params=_SC_PARAMS)
    return f(xt, srcw, dstw, zeros, ones)


def _sc_combine(x16, sp16, dg16, scal16):
    q = x16.shape[0]
    share = q // NW
    mesh = plsc.VectorSubcoreMesh(core_axis_name="c", subcore_axis_name="s")
    scratch = [pltpu.VMEM((share, 16), jnp.float32),
               pltpu.VMEM((share, 16), jnp.float32),
               pltpu.VMEM((share, 16), jnp.float32),
               pltpu.VMEM((share, 16), jnp.float32),
               pltpu.VMEM((share, 16), jnp.float32),
               pltpu.VMEM((share, 16), jnp.float32),
               pltpu.VMEM((8, 16), jnp.float32)]

    def body(x_h, sp_h, dg_h, scal_h, o_h, x_v, sa_v, sb_v, da_v, db_v,
             o_v, sc_v):
        cid = lax.axis_index("c")
        sid = lax.axis_index("s")
        row0 = (cid * NS + sid) * share
        pltpu.sync_copy(x_h.at[pl.ds(row0, share)], x_v)
        pltpu.sync_copy(sp_h.at[0, pl.ds(row0, share)], sa_v)
        pltpu.sync_copy(sp_h.at[1, pl.ds(row0, share)], sb_v)
        pltpu.sync_copy(dg_h.at[0, pl.ds(row0, share)], da_v)
        pltpu.sync_copy(dg_h.at[1, pl.ds(row0, share)], db_v)
        pltpu.sync_copy(scal_h, sc_v)
        k1 = sc_v[0]
        k2 = sc_v[1]
        sw = sc_v[2]
        nw = sc_v[3]
        b = sc_v[4]

        def step(i, carry):
            deg = da_v[i] + db_v[i]
            bits = lax.bitcast_convert_type(deg, jnp.int32)
            e = (bits >> 23) - 127
            mant = lax.bitcast_convert_type((bits & 0x007FFFFF) | 0x3F800000,
                                            jnp.float32)
            p = jnp.full_like(mant, _L2C[0])
            for c in _L2C[1:]:
                p = p * mant + c
            l2 = e.astype(jnp.float32) + p
            f1 = jnp.exp(k1 * l2)
            f2 = jnp.exp(k2 * l2)
            o_v[i] = sw * f1 * x_v[i] + nw * f2 * (sa_v[i] + sb_v[i]) + b
            return carry

        lax.fori_loop(0, share, step, 0)
        pltpu.sync_copy(o_v, o_h.at[pl.ds(row0, share)])

    f = pl.kernel(body, out_type=jax.ShapeDtypeStruct((q, 16), jnp.float32),
                  mesh=mesh, scratch_types=scratch, compiler_params=_SC_PARAMS)
    return f(x16, sp16, dg16, scal16)


def kernel(x, edge_index, alpha1, alpha2, gamma, bias):
    t_ch, n = x.shape
    e = edge_index.shape[1]
    n_layers = alpha1.shape[0]
    np_ = (n // 2048 + 1) * 2048
    q = np_ * CH // 16
    nwin = e // W

    srcw = edge_index[0].reshape(nwin, W)
    dstw = edge_index[1].reshape(nwin, W)
    xt = jnp.pad(x.T, ((0, np_ - n), (0, 0)))
    zeros = jnp.zeros((np_ // NS, CH), jnp.float32)
    ones = jnp.ones((W, CH), jnp.float32)

    out16 = xt.reshape(q, 16)
    dg16 = None
    for i in range(n_layers):
        tab = out16.reshape(np_, CH)
        if i == 0:
            sp, dgp = _sc_pass(tab, srcw, dstw, zeros, ones,
                               np_=np_, nwin=nwin, with_deg=True)
            dg16 = dgp.reshape(NC, q, 16)
        else:
            (sp,) = _sc_pass(tab, srcw, dstw, zeros, ones,
                             np_=np_, nwin=nwin, with_deg=False)
        a1 = alpha1[i, 0, 0]
        dp = jax.nn.sigmoid(gamma[i, 0, 0])
        sw = jnp.exp(a1)
        scal = jnp.stack([dp * LN2, (dp - 1.0) * LN2, sw, sw * jnp.tanh(a1),
                          bias[i, 0, 0], 0.0, 0.0, 0.0])
        scal16 = jnp.broadcast_to(scal[:, None], (8, 16))
        out16 = _sc_combine(out16, sp.reshape(NC, q, 16), dg16, scal16)
    return out16.reshape(np_, CH)[:n].T


# speedup vs baseline: 109.9693x; 1.8521x over previous
import jax
import jax.numpy as jnp
from jax import lax
from jax.experimental import pallas as pl
from jax.experimental.pallas import tpu as pltpu
from jax.experimental.pallas import tpu_sc as plsc

NC = 2
NS = 16
NW = NC * NS
W = 128
KJ = 16
CH = 4
R = 8
LN2 = 0.6931471805599453

_L2C = (0.0439286278, -0.409475586, 1.61017755, -3.52021884, 5.06975632,
        -2.79415368)

_SC_PARAMS = pltpu.CompilerParams(use_tc_tiling_on_sc=False)


def _sc_pass(xt8, srcw, dstw, zeros, e4, *, np_, nwin, with_deg):
    slab = np_ // NS
    base = nwin // NW
    rem = nwin % NW
    mesh = plsc.VectorSubcoreMesh(core_axis_name="c", subcore_axis_name="s")
    out_type = jax.ShapeDtypeStruct((NC, np_, R), jnp.float32)
    scratch = [pltpu.VMEM_SHARED((np_, R), jnp.float32),
               pltpu.VMEM_SHARED((np_, R), jnp.float32),
               pltpu.VMEM((KJ, W), jnp.int32),
               pltpu.VMEM((KJ, W), jnp.int32),
               pltpu.VMEM((KJ * W, R), jnp.float32),
               pltpu.VMEM((W, R), jnp.float32),
               pltpu.SemaphoreType.DMA,
               pltpu.SemaphoreType.DMA]

    def body(xt_h, srcw_h, dstw_h, zeros_h, e4_h, s_out, tab, acc,
             srcb, dstb, msg, e4_v, sem_g, sem_s):
        cid = lax.axis_index("c")
        sid = lax.axis_index("s")
        row0 = sid * slab
        pltpu.sync_copy(zeros_h, acc.at[pl.ds(row0, slab)])
        if with_deg:
            pltpu.sync_copy(e4_h, e4_v)
        pltpu.sync_copy(xt_h.at[pl.ds(row0, slab)], tab.at[pl.ds(row0, slab)])
        plsc.subcore_barrier()

        wid = cid * NS + sid
        wstart = wid * base + jnp.minimum(wid, rem)
        wcnt = base + jnp.where(wid < rem, 1, 0)
        nfull = wcnt // KJ

        def do_windows(nj):
            cps = []
            for j in range(nj):
                cps.append(pltpu.async_copy(
                    tab.at[srcb.at[j]], msg.at[pl.ds(j * W, W)], sem_g))
            for c in cps:
                c.wait()
            cps = []
            for j in range(nj):
                cps.append(pltpu.async_copy(
                    msg.at[pl.ds(j * W, W)], acc.at[dstb.at[j]], sem_s,
                    add=True))
                if with_deg:
                    cps.append(pltpu.async_copy(
                        e4_v, acc.at[srcb.at[j]], sem_s, add=True))
            for c in cps:
                c.wait()

        def step(g, carry):
            row = wstart + g * KJ
            pltpu.sync_copy(srcw_h.at[pl.ds(row, KJ)], srcb)
            pltpu.sync_copy(dstw_h.at[pl.ds(row, KJ)], dstb)
            do_windows(KJ)
            return carry

        lax.fori_loop(0, nfull, step, 0)

        def tail_step(t, carry):
            row = wstart + nfull * KJ + t
            pltpu.sync_copy(srcw_h.at[pl.ds(row, 1)], srcb.at[pl.ds(0, 1)])
            pltpu.sync_copy(dstw_h.at[pl.ds(row, 1)], dstb.at[pl.ds(0, 1)])
            do_windows(1)
            return carry

        lax.fori_loop(0, wcnt - nfull * KJ, tail_step, 0)

        plsc.subcore_barrier()
        pltpu.sync_copy(acc.at[pl.ds(row0, slab)],
                        s_out.at[cid, pl.ds(row0, slab)])

    f = pl.kernel(body, out_type=out_type, mesh=mesh,
                  scratch_types=scratch, compiler_params=_SC_PARAMS)
    return f(xt8, srcw, dstw, zeros, e4)


def _sc_combine(x16, sp16, dg16, scal16, *, deg_from_s):
    q8 = x16.shape[0]
    share = q8 // NW
    ck = share // 2
    mesh = plsc.VectorSubcoreMesh(core_axis_name="c", subcore_axis_name="s")
    bufs = 4 if deg_from_s else 6
    scratch = ([pltpu.VMEM((ck, 16), jnp.float32)] * bufs
               + [pltpu.VMEM((8, 16), jnp.float32)])

    def body(x_h, sp_h, dg_h, scal_h, o_h, *rest):
        if deg_from_s:
            x_v, sa_v, sb_v, o_v, sc_v = rest
            da_v = db_v = None
        else:
            x_v, sa_v, sb_v, da_v, db_v, o_v, sc_v = rest
        cid = lax.axis_index("c")
        sid = lax.axis_index("s")
        row0 = (cid * NS + sid) * share
        pltpu.sync_copy(scal_h, sc_v)
        k1 = sc_v[0]
        k2 = sc_v[1]
        sw = sc_v[2]
        nw = sc_v[3]
        b = sc_v[4]
        iota = lax.iota(jnp.int32, 16)
        bidx = (iota & 8) + 4

        for k in range(2):
            base = row0 + k * ck
            pltpu.sync_copy(x_h.at[pl.ds(base, ck)], x_v)
            pltpu.sync_copy(sp_h.at[0, pl.ds(base, ck)], sa_v)
            pltpu.sync_copy(sp_h.at[1, pl.ds(base, ck)], sb_v)
            if not deg_from_s:
                pltpu.sync_copy(dg_h.at[0, pl.ds(base, ck)], da_v)
                pltpu.sync_copy(dg_h.at[1, pl.ds(base, ck)], db_v)

            def step(i, carry):
                s = sa_v[i] + sb_v[i]
                dsrc = s if deg_from_s else da_v[i] + db_v[i]
                deg = jnp.take(dsrc, bidx)
                bits = lax.bitcast_convert_type(deg, jnp.int32)
                e = (bits >> 23) - 127
                mant = lax.bitcast_convert_type(
                    (bits & 0x007FFFFF) | 0x3F800000, jnp.float32)
                p = jnp.full_like(mant, _L2C[0])
                for c in _L2C[1:]:
                    p = p * mant + c
                l2 = e.astype(jnp.float32) + p
                f1 = jnp.exp(k1 * l2)
                f2 = jnp.exp(k2 * l2)
                o_v[i] = sw * f1 * x_v[i] + nw * f2 * s + b
                return carry

            lax.fori_loop(0, ck, step, 0)
            pltpu.sync_copy(o_v, o_h.at[pl.ds(base, ck)])

    f = pl.kernel(body, out_type=jax.ShapeDtypeStruct((q8, 16), jnp.float32),
                  mesh=mesh, scratch_types=scratch, compiler_params=_SC_PARAMS)
    return f(x16, sp16, dg16, scal16)


def kernel(x, edge_index, alpha1, alpha2, gamma, bias):
    t_ch, n = x.shape
    e = edge_index.shape[1]
    n_layers = alpha1.shape[0]
    np_ = (n // 2048 + 1) * 2048
    q8 = np_ // 2
    nwin = e // W

    srcw = edge_index[0].reshape(nwin, W)
    dstw = edge_index[1].reshape(nwin, W)
    xt8 = jnp.pad(x.T, ((0, np_ - n), (0, R - CH)))
    zeros = jnp.zeros((np_ // NS, R), jnp.float32)
    e4 = jnp.zeros((W, R), jnp.float32).at[:, CH].set(1.0)

    out16 = xt8.reshape(q8, 16)
    dg16 = None
    for i in range(n_layers):
        sp = _sc_pass(out16.reshape(np_, R), srcw, dstw, zeros, e4,
                      np_=np_, nwin=nwin, with_deg=(i == 0))
        sp16 = sp.reshape(NC, q8, 16)
        if i == 0:
            dg16 = sp16
        a1 = alpha1[i, 0, 0]
        dp = jax.nn.sigmoid(gamma[i, 0, 0])
        sw = jnp.exp(a1)
        scal = jnp.stack([dp * LN2, (dp - 1.0) * LN2, sw, sw * jnp.tanh(a1),
                          bias[i, 0, 0], 0.0, 0.0, 0.0])
        scal16 = jnp.broadcast_to(scal[:, None], (8, 16))
        out16 = _sc_combine(out16, sp16, dg16, scal16, deg_from_s=(i == 0))
    return out16.reshape(np_, R)[:n, :CH].T
